# baseline (speedup 1.0000x reference)
import functools

import jax
import jax.numpy as jnp
from jax import lax
from jax.experimental import pallas as pl
from jax.experimental.pallas import tpu as pltpu
from jax.experimental.pallas import tpu_sc as plsc

D_MODEL = 1024
NUM_CORES = 2
NUM_SUBCORES = 16
NUM_WORKERS = NUM_CORES * NUM_SUBCORES
CHUNK = 32
NBUF = 2


def _emb_body(idx_hbm, table_hbm, out_hbm, idx_v, rows_v,
              gsem0, gsem1, wsem0, wsem1, b_per_w):
    gsems = (gsem0, gsem1)
    wsems = (wsem0, wsem1)
    wid = lax.axis_index("s") * NUM_CORES + lax.axis_index("c")
    base = wid * b_per_w
    rounds = b_per_w // CHUNK // NBUF
    pltpu.sync_copy(idx_hbm.at[pl.ds(base, b_per_w)], idx_v)

    def g_desc(i, b):
        return pltpu.make_async_copy(
            table_hbm.at[idx_v.at[pl.ds(i * CHUNK, CHUNK)]],
            rows_v.at[b], gsems[b])

    def w_desc(i, b):
        return pltpu.make_async_copy(
            rows_v.at[b], out_hbm.at[pl.ds(base + i * CHUNK, CHUNK)],
            wsems[b])

    for b in range(NBUF):
        g_desc(b, b).start()

    def round_(j, carry):
        for b in range(NBUF):
            i = j * NBUF + b
            g_desc(i, b).wait()
            w_desc(i, b).start()
        for b in range(NBUF):
            i = j * NBUF + b
            w_desc(i, b).wait()
            g_desc(i + NBUF, b).start()
        return carry

    lax.fori_loop(0, rounds - 1, round_, 0)

    jlast = rounds - 1
    for b in range(NBUF):
        i = jlast * NBUF + b
        g_desc(i, b).wait()
        w_desc(i, b).start()
    for b in range(NBUF):
        w_desc(jlast * NBUF + b, b).wait()


@functools.partial(jax.jit, static_argnames=())
def _emb_lookup(x_flat, table):
    b = x_flat.shape[0]
    b_per_w = b // NUM_WORKERS
    mesh = plsc.VectorSubcoreMesh(core_axis_name="c", subcore_axis_name="s")
    fn = pl.kernel(
        functools.partial(_emb_body, b_per_w=b_per_w),
        mesh=mesh,
        out_type=jax.ShapeDtypeStruct((b, D_MODEL), jnp.float32),
        scratch_types=[
            pltpu.VMEM((b_per_w,), jnp.int32),
            pltpu.VMEM((NBUF, CHUNK, D_MODEL), jnp.float32),
            pltpu.SemaphoreType.DMA,
            pltpu.SemaphoreType.DMA,
            pltpu.SemaphoreType.DMA,
            pltpu.SemaphoreType.DMA,
        ],
    )
    return fn(x_flat, table)


def kernel(x, embedding_weight):
    batch, seq = x.shape
    out = _emb_lookup(x.reshape(batch * seq).astype(jnp.int32), embedding_weight)
    return out.reshape(batch, seq, D_MODEL)


# speedup vs baseline: 1.6525x; 1.6525x over previous
import functools

import jax
import jax.numpy as jnp
from jax import lax
from jax.experimental import pallas as pl
from jax.experimental.pallas import tpu as pltpu
from jax.experimental.pallas import tpu_sc as plsc

D_MODEL = 1024
VOCAB = 256
NUM_CORES = 2
NUM_SUBCORES = 16
NUM_WORKERS = NUM_CORES * NUM_SUBCORES
SUB = 256


def _emb_body(idx_hbm, table_hbm, out_hbm, idx_s, table_s, idx_sh, cp_sem,
              b_per_w):
    sid = lax.axis_index("s")
    wid = sid * NUM_CORES + lax.axis_index("c")
    base = wid * b_per_w

    @pl.when(sid == 0)
    def _stage():
        pltpu.sync_copy(table_hbm, table_s)
        pltpu.sync_copy(idx_hbm, idx_sh)

    plsc.subcore_barrier()

    def outer(c, carry):
        off = base + c * SUB
        pltpu.sync_copy(idx_sh.at[pl.ds(off, SUB)], idx_s)

        def inner(t, carry2):
            s = idx_s[t]
            pltpu.async_copy(
                table_s.at[pl.ds(s, 1)], out_hbm.at[pl.ds(off + t, 1)], cp_sem
            )
            return carry2

        lax.fori_loop(0, SUB, inner, 0)
        pltpu.make_async_copy(
            table_s.at[pl.ds(0, SUB)], out_hbm.at[pl.ds(off, SUB)], cp_sem
        ).wait()
        return carry

    lax.fori_loop(0, b_per_w // SUB, outer, 0)


@functools.partial(jax.jit, static_argnames=())
def _emb_lookup(x_flat, table):
    b = x_flat.shape[0]
    b_per_w = b // NUM_WORKERS
    mesh = plsc.VectorSubcoreMesh(core_axis_name="c", subcore_axis_name="s")
    fn = pl.kernel(
        functools.partial(_emb_body, b_per_w=b_per_w),
        mesh=mesh,
        out_type=jax.ShapeDtypeStruct((b, D_MODEL), jnp.float32),
        scratch_types=[
            pltpu.SMEM((SUB,), jnp.int32),
            pltpu.VMEM_SHARED((VOCAB, D_MODEL), jnp.float32),
            pltpu.VMEM_SHARED((b,), jnp.int32),
            pltpu.SemaphoreType.DMA,
        ],
    )
    return fn(x_flat, table)


def kernel(x, embedding_weight):
    batch, seq = x.shape
    out = _emb_lookup(x.reshape(batch * seq).astype(jnp.int32), embedding_weight)
    return out.reshape(batch, seq, D_MODEL)


# speedup vs baseline: 1.6634x; 1.0066x over previous
import functools

import jax
import jax.numpy as jnp
from jax import lax
from jax.experimental import pallas as pl
from jax.experimental.pallas import tpu as pltpu
from jax.experimental.pallas import tpu_sc as plsc

D_MODEL = 1024
VOCAB = 256
NUM_CORES = 2
NUM_SUBCORES = 16
NUM_WORKERS = NUM_CORES * NUM_SUBCORES
UNROLL = 4


def _emb_body(idx_hbm, table_hbm, out_hbm, idx_s, table_s, idx_sh, cp_sem,
              b_per_w):
    sid = lax.axis_index("s")
    wid = sid * NUM_CORES + lax.axis_index("c")
    base = wid * b_per_w
    rows_per_tile = VOCAB // NUM_SUBCORES

    pltpu.sync_copy(table_hbm.at[pl.ds(sid * rows_per_tile, rows_per_tile)],
                    table_s.at[pl.ds(sid * rows_per_tile, rows_per_tile)])
    pltpu.sync_copy(idx_hbm.at[pl.ds(base, b_per_w)],
                    idx_sh.at[pl.ds(base, b_per_w)])
    pltpu.sync_copy(idx_sh.at[pl.ds(base, b_per_w)], idx_s)
    plsc.subcore_barrier()

    def body(j, carry):
        for k in range(UNROLL):
            t = j * UNROLL + k
            s = idx_s[t]
            pltpu.async_copy(
                table_s.at[pl.ds(s, 1)], out_hbm.at[pl.ds(base + t, 1)],
                cp_sem)
        return carry

    lax.fori_loop(0, b_per_w // UNROLL, body, 0)

    pltpu.make_async_copy(
        out_hbm.at[pl.ds(base, b_per_w)], out_hbm.at[pl.ds(base, b_per_w)],
        cp_sem).wait()


@functools.partial(jax.jit, static_argnames=())
def _emb_lookup(x_flat, table):
    b = x_flat.shape[0]
    b_per_w = b // NUM_WORKERS
    mesh = plsc.VectorSubcoreMesh(core_axis_name="c", subcore_axis_name="s")
    fn = pl.kernel(
        functools.partial(_emb_body, b_per_w=b_per_w),
        mesh=mesh,
        out_type=jax.ShapeDtypeStruct((b, D_MODEL), jnp.float32),
        scratch_types=[
            pltpu.SMEM((b_per_w,), jnp.int32),
            pltpu.VMEM_SHARED((VOCAB, D_MODEL), jnp.float32),
            pltpu.VMEM_SHARED((b,), jnp.int32),
            pltpu.SemaphoreType.DMA,
        ],
    )
    return fn(x_flat, table)


def kernel(x, embedding_weight):
    batch, seq = x.shape
    out = _emb_lookup(x.reshape(batch * seq).astype(jnp.int32), embedding_weight)
    return out.reshape(batch, seq, D_MODEL)
